# baseline (speedup 1.0000x reference)
import functools

import jax
import jax.numpy as jnp
from jax import lax
from jax.experimental import pallas as pl
from jax.experimental.pallas import tpu as pltpu
from jax.experimental.pallas import tpu_sc as plsc

_B = 4096
_H = 64
_NC = 2
_NS = 16
_NW = _NC * _NS
_BPW = _B // _NW

_mesh = plsc.VectorSubcoreMesh(core_axis_name="c", subcore_axis_name="s")


@functools.partial(
    pl.kernel,
    mesh=_mesh,
    out_type=jax.ShapeDtypeStruct((_B, 8, _H), jnp.float32),
    scratch_types=[
        pltpu.VMEM((_BPW,), jnp.int32),
        pltpu.VMEM((_BPW,), jnp.int32),
        pltpu.VMEM((_BPW,), jnp.int32),
        pltpu.VMEM((_BPW,), jnp.int32),
        pltpu.VMEM((_BPW, _H), jnp.float32),
        pltpu.VMEM((_BPW, _H), jnp.float32),
        pltpu.VMEM((_BPW, _H), jnp.float32),
        pltpu.VMEM((_BPW, _H), jnp.float32),
        pltpu.VMEM((_BPW, _H), jnp.float32),
        pltpu.VMEM((_BPW, _H), jnp.float32),
        pltpu.VMEM((_BPW, _H), jnp.float32),
        pltpu.VMEM((_BPW, _H), jnp.float32),
        pltpu.SemaphoreType.DMA,
        pltpu.SemaphoreType.DMA,
    ],
    compiler_params=pltpu.CompilerParams(use_tc_tiling_on_sc=False),
)
def _mdp_embed(s0, s1, s2, s3, i0, i1, i2, i3, table, out,
               x0, x1, x2, x3, sb0, sb1, sb2, sb3, gb0, gb1, gb2, gb3,
               lsem, ssem):
    wid = lax.axis_index("s") * _NC + lax.axis_index("c")
    base = wid * _BPW
    states = (s0, s1, s2, s3)
    idx_hbm = (i0, i1, i2, i3)
    idx = (x0, x1, x2, x3)
    sbufs = (sb0, sb1, sb2, sb3)
    gbufs = (gb0, gb1, gb2, gb3)

    for i in range(4):
        pltpu.sync_copy(idx_hbm[i].at[pl.ds(base, _BPW)], idx[i])

    loads = []
    for i in range(4):
        loads.append(pltpu.async_copy(states[i].at[pl.ds(base, _BPW)], sbufs[i], lsem))
        loads.append(pltpu.async_copy(table.at[idx[i]], gbufs[i], lsem))
    for c in loads:
        c.wait()

    stores = []
    for i in range(4):
        stores.append(pltpu.async_copy(sbufs[i], out.at[pl.ds(base, _BPW), 2 * i], ssem))
        stores.append(pltpu.async_copy(gbufs[i], out.at[pl.ds(base, _BPW), 2 * i + 1], ssem))
    for c in stores:
        c.wait()


def kernel(s0, a0, s1, a1, s2, a2, s3, a3, table):
    i0 = a0.reshape(-1).astype(jnp.int32)
    i1 = a1.reshape(-1).astype(jnp.int32)
    i2 = a2.reshape(-1).astype(jnp.int32)
    i3 = a3.reshape(-1).astype(jnp.int32)
    return _mdp_embed(s0, s1, s2, s3, i0, i1, i2, i3, table)
